# baseline (speedup 1.0000x reference)
import functools

import jax
import jax.numpy as jnp
from jax import lax
from jax.experimental import pallas as pl
from jax.experimental.pallas import tpu as pltpu
from jax.experimental.pallas import tpu_sc as plsc

D_MODEL = 128
NUM_CORES = 2
NUM_SUBCORES = 16
NUM_WORKERS = NUM_CORES * NUM_SUBCORES
CHUNK = 64
NBUF = 8
LAG = 4


@functools.partial(jax.jit, static_argnums=(2,))
def _gather_rows(pe, idx_flat, n_rows):
    bpw = n_rows // NUM_WORKERS
    n_chunks = bpw // CHUNK
    assert n_rows % NUM_WORKERS == 0 and bpw % CHUNK == 0
    assert n_chunks % NBUF == 0 and n_chunks >= 2 * NBUF

    mesh = plsc.VectorSubcoreMesh(core_axis_name="c", subcore_axis_name="s")

    @functools.partial(
        pl.kernel,
        mesh=mesh,
        out_type=jax.ShapeDtypeStruct((n_rows, D_MODEL), jnp.float32),
        scratch_types=[
            pltpu.VMEM((bpw,), jnp.int32),
        ]
        + [pltpu.VMEM((CHUNK, D_MODEL), jnp.float32) for _ in range(NBUF)]
        + [pltpu.SemaphoreType.DMA for _ in range(2 * NBUF)],
    )
    def gather_kernel(pe_hbm, idx_hbm, out_hbm, idx_v, *bufs_and_sems):
        rows = bufs_and_sems[:NBUF]
        gsem = bufs_and_sems[NBUF:2 * NBUF]
        osem = bufs_and_sems[2 * NBUF:]

        wid = lax.axis_index("s") * NUM_CORES + lax.axis_index("c")
        base = wid * bpw
        pltpu.sync_copy(idx_hbm.at[pl.ds(base, bpw)], idx_v)

        def gather_copy(j, b):
            return pltpu.make_async_copy(
                pe_hbm.at[idx_v.at[pl.ds(j * CHUNK, CHUNK)]], rows[b], gsem[b])

        def out_copy(j, b):
            return pltpu.make_async_copy(
                rows[b], out_hbm.at[pl.ds(base + j * CHUNK, CHUNK)], osem[b])

        for j in range(NBUF):
            gather_copy(j, j).start()
            if j >= LAG:
                jo = j - LAG
                gather_copy(jo, jo).wait()
                out_copy(jo, jo).start()

        def body(i, carry):
            for b in range(NBUF):
                j = NBUF + i * NBUF + b
                bo = (b + NBUF - LAG) % NBUF
                out_copy(j - NBUF, b).wait()
                gather_copy(j, b).start()
                gather_copy(j - LAG, bo).wait()
                out_copy(j - LAG, bo).start()
            return carry

        lax.fori_loop(0, n_chunks // NBUF - 1, body, 0)

        for j in range(n_chunks - LAG, n_chunks):
            b = j % NBUF
            gather_copy(j, b).wait()
            out_copy(j, b).start()
        for j in range(n_chunks - NBUF, n_chunks):
            out_copy(j, j % NBUF).wait()

    return gather_kernel(pe, idx_flat)


def kernel(t, pe):
    b, s = t.shape
    idx_t = jnp.swapaxes(t, 0, 1).reshape(-1)
    out = _gather_rows(pe, idx_t, b * s)
    return out.reshape(s, b, D_MODEL).transpose(1, 0, 2)


# speedup vs baseline: 1.0026x; 1.0026x over previous
import functools

import jax
import jax.numpy as jnp
from jax import lax
from jax.experimental import pallas as pl
from jax.experimental.pallas import tpu as pltpu
from jax.experimental.pallas import tpu_sc as plsc

D_MODEL = 128
NUM_CORES = 2
NUM_SUBCORES = 16
NUM_WORKERS = NUM_CORES * NUM_SUBCORES
CHUNK = 128
NBUF = 4
LAG = 2


@functools.partial(jax.jit, static_argnums=(2,))
def _gather_rows(pe, idx_flat, n_rows):
    bpw = n_rows // NUM_WORKERS
    n_chunks = bpw // CHUNK
    assert n_rows % NUM_WORKERS == 0 and bpw % CHUNK == 0
    assert n_chunks % NBUF == 0 and n_chunks >= 2 * NBUF

    mesh = plsc.VectorSubcoreMesh(core_axis_name="c", subcore_axis_name="s")

    @functools.partial(
        pl.kernel,
        mesh=mesh,
        out_type=jax.ShapeDtypeStruct((n_rows, D_MODEL), jnp.float32),
        scratch_types=[
            pltpu.VMEM((bpw,), jnp.int32),
        ]
        + [pltpu.VMEM((CHUNK, D_MODEL), jnp.float32) for _ in range(NBUF)]
        + [pltpu.SemaphoreType.DMA for _ in range(2 * NBUF)],
    )
    def gather_kernel(pe_hbm, idx_hbm, out_hbm, idx_v, *bufs_and_sems):
        rows = bufs_and_sems[:NBUF]
        gsem = bufs_and_sems[NBUF:2 * NBUF]
        osem = bufs_and_sems[2 * NBUF:]

        wid = lax.axis_index("s") * NUM_CORES + lax.axis_index("c")
        base = wid * bpw
        pltpu.sync_copy(idx_hbm.at[pl.ds(base, bpw)], idx_v)

        def gather_copy(j, b):
            return pltpu.make_async_copy(
                pe_hbm.at[idx_v.at[pl.ds(j * CHUNK, CHUNK)]], rows[b], gsem[b])

        def out_copy(j, b):
            return pltpu.make_async_copy(
                rows[b], out_hbm.at[pl.ds(base + j * CHUNK, CHUNK)], osem[b])

        for j in range(NBUF):
            gather_copy(j, j).start()
            if j >= LAG:
                jo = j - LAG
                gather_copy(jo, jo).wait()
                out_copy(jo, jo).start()

        def body(i, carry):
            for b in range(NBUF):
                j = NBUF + i * NBUF + b
                bo = (b + NBUF - LAG) % NBUF
                out_copy(j - NBUF, b).wait()
                gather_copy(j, b).start()
                gather_copy(j - LAG, bo).wait()
                out_copy(j - LAG, bo).start()
            return carry

        lax.fori_loop(0, n_chunks // NBUF - 1, body, 0)

        for j in range(n_chunks - LAG, n_chunks):
            b = j % NBUF
            gather_copy(j, b).wait()
            out_copy(j, b).start()
        for j in range(n_chunks - NBUF, n_chunks):
            out_copy(j, j % NBUF).wait()

    return gather_kernel(pe, idx_flat)


def kernel(t, pe):
    b, s = t.shape
    idx_t = jnp.swapaxes(t, 0, 1).reshape(-1)
    out = _gather_rows(pe, idx_t, b * s)
    return out.reshape(s, b, D_MODEL).transpose(1, 0, 2)
